# Initial kernel scaffold; baseline (speedup 1.0000x reference)
import jax
import jax.numpy as jnp
from jax.experimental import pallas as pl


def kernel(x, edge_index, mask, labels, W1, b1, W2, b2, Wo, bo):
    raise NotImplementedError("write your pallas kernel here")


# speedup vs baseline: 15.7232x; 15.7232x over previous
import functools

import jax
import jax.numpy as jnp
from jax import lax
from jax.experimental import pallas as pl
from jax.experimental.pallas import tpu as pltpu
from jax.experimental.pallas import tpu_sc as plsc

N = 10000
E = 320000
D_IN = 128
H = 256
D_EMB = 128

NPAD = 10240
NC, NS = 2, 16
NW = NC * NS
CH = 128
EPW = 10112
EP = NW * EPW
DUMMY = 10200
RPT = NPAD // NS
BS = 512
GRID = NPAD // BS


def _sc_prop(D, gather):
    mesh = plsc.VectorSubcoreMesh(core_axis_name="c", subcore_axis_name="s")
    out_type = jax.ShapeDtypeStruct((NC, NPAD, D), jnp.float32)
    cparams = pltpu.CompilerParams(use_tc_tiling_on_sc=(D == D_IN))

    if gather:
        scratch = [
            pltpu.VMEM((CH,), jnp.int32),
            pltpu.VMEM((CH,), jnp.int32),
            pltpu.VMEM((CH, D), jnp.float32),
            pltpu.SemaphoreType.DMA,
            pltpu.VMEM_SHARED((NPAD, D), jnp.float32),
        ]

        @functools.partial(pl.kernel, out_type=out_type, mesh=mesh,
                           scratch_types=scratch, compiler_params=cparams)
        def k(src_h, dst_h, tab_h, z_h, out_h, src_v, dst_v, rows_v, sem, acc):
            cid = lax.axis_index("c")
            sid = lax.axis_index("s")
            r0 = sid * RPT
            pltpu.sync_copy(z_h, acc.at[pl.ds(r0, RPT)])
            plsc.subcore_barrier()
            base = (sid * NC + cid) * EPW

            def step(i, carry):
                off = base + i * CH
                pltpu.sync_copy(src_h.at[pl.ds(off, CH)], src_v)
                pltpu.sync_copy(dst_h.at[pl.ds(off, CH)], dst_v)
                pltpu.async_copy(tab_h.at[src_v], rows_v, sem).wait()
                pltpu.sync_copy(rows_v, acc.at[dst_v], add=True)
                return carry

            lax.fori_loop(0, EPW // CH, step, 0)
            plsc.subcore_barrier()
            pltpu.sync_copy(acc.at[pl.ds(r0, RPT)],
                            out_h.at[cid, pl.ds(r0, RPT)])

        return k

    scratch = [
        pltpu.VMEM((CH,), jnp.int32),
        pltpu.VMEM((CH, D), jnp.float32),
        pltpu.VMEM_SHARED((NPAD, D), jnp.float32),
    ]

    @functools.partial(pl.kernel, out_type=out_type, mesh=mesh,
                       scratch_types=scratch, compiler_params=cparams)
    def k(dst_h, ones_h, z_h, out_h, dst_v, rows_v, acc):
        cid = lax.axis_index("c")
        sid = lax.axis_index("s")
        r0 = sid * RPT
        pltpu.sync_copy(z_h, acc.at[pl.ds(r0, RPT)])
        pltpu.sync_copy(ones_h, rows_v)
        plsc.subcore_barrier()
        base = (sid * NC + cid) * EPW

        def step(i, carry):
            off = base + i * CH
            pltpu.sync_copy(dst_h.at[pl.ds(off, CH)], dst_v)
            pltpu.sync_copy(rows_v, acc.at[dst_v], add=True)
            return carry

        lax.fori_loop(0, EPW // CH, step, 0)
        plsc.subcore_barrier()
        pltpu.sync_copy(acc.at[pl.ds(r0, RPT)], out_h.at[cid, pl.ds(r0, RPT)])

    return k


_sc_deg = _sc_prop(8, gather=False)
_sc_prop128 = _sc_prop(D_IN, gather=True)
_sc_prop8 = _sc_prop(8, gather=True)


def _tc_scale_body(deg_ref, x_ref, dis_ref, xs_ref):
    deg = deg_ref[0][:, :1] + deg_ref[1][:, :1] + 1.0
    dis = lax.rsqrt(deg)
    dis_ref[...] = dis
    xs_ref[...] = x_ref[...] * dis


def _tc_scale(deg2, x):
    return pl.pallas_call(
        _tc_scale_body,
        grid=(GRID,),
        in_specs=[
            pl.BlockSpec((NC, BS, 8), lambda i: (0, i, 0)),
            pl.BlockSpec((BS, D_IN), lambda i: (i, 0)),
        ],
        out_specs=[
            pl.BlockSpec((BS, 1), lambda i: (i, 0)),
            pl.BlockSpec((BS, D_IN), lambda i: (i, 0)),
        ],
        out_shape=[
            jax.ShapeDtypeStruct((NPAD, 1), jnp.float32),
            jax.ShapeDtypeStruct((NPAD, D_IN), jnp.float32),
        ],
    )(deg2, x)


def _tc_mlp_body(y2_ref, xs_ref, dis_ref, w1_ref, b1_ref, w2_ref, wo_ref,
                 qs_ref):
    dis = dis_ref[...]
    y = dis * (y2_ref[0] + y2_ref[1] + xs_ref[...])
    h = jnp.dot(y, w1_ref[...], preferred_element_type=jnp.float32)
    h = jnp.maximum(h + b1_ref[...], 0.0)
    v = jnp.dot(w2_ref[...], wo_ref[...], preferred_element_type=jnp.float32)
    q = jnp.dot(h, v, preferred_element_type=jnp.float32)
    qs_ref[...] = jnp.broadcast_to(dis * q, (BS, 8))


def _tc_mlp(y2, xs, dis, W1, b1, W2, Wo):
    return pl.pallas_call(
        _tc_mlp_body,
        grid=(GRID,),
        in_specs=[
            pl.BlockSpec((NC, BS, D_IN), lambda i: (0, i, 0)),
            pl.BlockSpec((BS, D_IN), lambda i: (i, 0)),
            pl.BlockSpec((BS, 1), lambda i: (i, 0)),
            pl.BlockSpec((D_IN, H), lambda i: (0, 0)),
            pl.BlockSpec((1, H), lambda i: (0, 0)),
            pl.BlockSpec((H, D_EMB), lambda i: (0, 0)),
            pl.BlockSpec((D_EMB, 1), lambda i: (0, 0)),
        ],
        out_specs=pl.BlockSpec((BS, 8), lambda i: (i, 0)),
        out_shape=jax.ShapeDtypeStruct((NPAD, 8), jnp.float32),
    )(y2, xs, dis, W1, b1, W2, Wo)


def _tc_loss_body(p2_ref, qs_ref, dis_ref, lab_ref, mask_ref, b2_ref, wo_ref,
                  bo_ref, out_ref, acc_ref):
    i = pl.program_id(0)
    c = jnp.dot(b2_ref[...], wo_ref[...],
                preferred_element_type=jnp.float32) + bo_ref[...]
    pred = dis_ref[...] * (p2_ref[0][:, :1] + p2_ref[1][:, :1]
                           + qs_ref[:, :1]) + c
    t = lab_ref[...]
    per = jnp.maximum(pred, 0.0) - pred * t + jnp.log1p(jnp.exp(-jnp.abs(pred)))
    m = mask_ref[...]
    bnum = jnp.sum(per * m)
    bden = jnp.sum(m)

    @pl.when(i == 0)
    def _():
        acc_ref[0] = bnum
        acc_ref[1] = bden

    @pl.when(i > 0)
    def _():
        acc_ref[0] = acc_ref[0] + bnum
        acc_ref[1] = acc_ref[1] + bden

    @pl.when(i == pl.num_programs(0) - 1)
    def _():
        out_ref[...] = (acc_ref[0] / acc_ref[1]).reshape(1, 1)


def _tc_loss(p2, qs, dis, lab, maskf, b2, Wo, bo):
    return pl.pallas_call(
        _tc_loss_body,
        grid=(GRID,),
        in_specs=[
            pl.BlockSpec((NC, BS, 8), lambda i: (0, i, 0)),
            pl.BlockSpec((BS, 8), lambda i: (i, 0)),
            pl.BlockSpec((BS, 1), lambda i: (i, 0)),
            pl.BlockSpec((BS, 1), lambda i: (i, 0)),
            pl.BlockSpec((BS, 1), lambda i: (i, 0)),
            pl.BlockSpec((1, D_EMB), lambda i: (0, 0)),
            pl.BlockSpec((D_EMB, 1), lambda i: (0, 0)),
            pl.BlockSpec((1, 1), lambda i: (0, 0)),
        ],
        out_specs=pl.BlockSpec((1, 1), lambda i: (0, 0)),
        out_shape=jax.ShapeDtypeStruct((1, 1), jnp.float32),
        scratch_shapes=[pltpu.SMEM((2,), jnp.float32)],
    )(p2, qs, dis, lab, maskf, b2, Wo, bo)


def kernel(x, edge_index, mask, labels, W1, b1, W2, b2, Wo, bo):
    src = edge_index[0]
    dst = edge_index[1]
    pad = EP - E
    src_p = jnp.concatenate([src, jnp.zeros((pad,), jnp.int32)])
    dst_p = jnp.concatenate([dst, jnp.full((pad,), DUMMY, jnp.int32)])
    x_p = jnp.pad(x, ((0, NPAD - N), (0, 0)))
    ones8 = jnp.ones((CH, 8), jnp.float32)
    z8 = jnp.zeros((RPT, 8), jnp.float32)
    z128 = jnp.zeros((RPT, D_IN), jnp.float32)

    deg2 = _sc_deg(dst_p, ones8, z8)
    dis, xs = _tc_scale(deg2, x_p)
    y2 = _sc_prop128(src_p, dst_p, xs, z128)
    qs = _tc_mlp(y2, xs, dis, W1, b1.reshape(1, H), W2, Wo)
    p2 = _sc_prop8(src_p, dst_p, qs, z8)

    lab_p = jnp.pad(labels, ((0, NPAD - N), (0, 0)))
    mf = jnp.pad(mask.astype(jnp.float32).reshape(N, 1),
                 ((0, NPAD - N), (0, 0)))
    loss = _tc_loss(p2, qs, dis, lab_p, mf, b2.reshape(1, D_EMB), Wo,
                    bo.reshape(1, 1))
    return loss.reshape(())


# speedup vs baseline: 16.2571x; 1.0340x over previous
import functools

import jax
import jax.numpy as jnp
from jax import lax
from jax.experimental import pallas as pl
from jax.experimental.pallas import tpu as pltpu
from jax.experimental.pallas import tpu_sc as plsc

N = 10000
E = 320000
D_IN = 128
H = 256
D_EMB = 128

NPAD = 10240
NC, NS = 2, 16
NW = NC * NS
CH = 128
NCH = 80
EPW = NCH * CH
EP = NW * EPW
DUMMY = 10200
RPT = NPAD // NS
BS = 512
GRID = NPAD // BS


def _sc_prop(D, gather, PD=2):
    mesh = plsc.VectorSubcoreMesh(core_axis_name="c", subcore_axis_name="s")
    out_type = jax.ShapeDtypeStruct((NC, NPAD, D), jnp.float32)
    cparams = pltpu.CompilerParams(use_tc_tiling_on_sc=(D == D_IN))

    if gather:
        scratch = [
            pltpu.VMEM((NCH, CH), jnp.int32),
            [pltpu.VMEM((CH,), jnp.int32) for _ in range(4)],
            pltpu.VMEM((2, CH, D), jnp.float32),
            [pltpu.SemaphoreType.DMA for _ in range(4)],
            pltpu.VMEM_SHARED((NPAD, D), jnp.float32),
        ]
        NIT = NCH // 4
        ROWB = CH * D * 4

        @functools.partial(pl.kernel, out_type=out_type, mesh=mesh,
                           scratch_types=scratch, compiler_params=cparams)
        def k(pk_h, tab_h, z_h, out_h, pk_l, slots, rows, sems, acc):
            cid = lax.axis_index("c")
            sid = lax.axis_index("s")
            wid = sid * NC + cid
            r0 = sid * RPT
            s0s, s0d, s1s, s1d = slots
            sg0, sg1, ss0, ss1 = sems
            pltpu.sync_copy(z_h, acc.at[pl.ds(r0, RPT)])
            pltpu.sync_copy(pk_h.at[wid], pk_l)
            plsc.subcore_barrier()

            def unpack(c, ssrc, sdst):
                for j in range(CH // 16):
                    v = pk_l[c, pl.ds(j * 16, 16)]
                    ssrc[pl.ds(j * 16, 16)] = lax.shift_right_logical(v, 16)
                    sdst[pl.ds(j * 16, 16)] = lax.bitwise_and(v, 0xFFFF)

            def g_start(ssrc, b, sem):
                pltpu.async_copy(tab_h.at[ssrc], rows.at[b], sem)

            def g_wait(ssrc, b, sem):
                pltpu.make_async_copy(tab_h.at[ssrc], rows.at[b], sem).wait()

            def s_start(sdst, b, sem):
                pltpu.async_copy(rows.at[b], acc.at[sdst], sem, add=True)

            def s_wait(sdst, b, sem):
                pltpu.make_async_copy(rows.at[b], acc.at[sdst], sem).wait()

            unpack(0, s0s, s0d)
            unpack(1, s1s, s1d)
            g_start(s0s, 0, sg0)
            g_start(s1s, 1, sg1)

            def step(g, carry):
                c0 = g * 4
                g_wait(s0s, 0, sg0)
                s_start(s0d, 0, ss0)
                g_wait(s1s, 1, sg1)
                s_start(s1d, 1, ss1)
                s_wait(s0d, 0, ss0)
                unpack(c0 + 2, s0s, s0d)
                g_start(s0s, 0, sg0)
                s_wait(s1d, 1, ss1)
                unpack(c0 + 3, s1s, s1d)
                g_start(s1s, 1, sg1)
                g_wait(s0s, 0, sg0)
                s_start(s0d, 0, ss0)
                g_wait(s1s, 1, sg1)
                s_start(s1d, 1, ss1)

                @pl.when(g < NIT - 1)
                def _():
                    s_wait(s0d, 0, ss0)
                    unpack(c0 + 4, s0s, s0d)
                    g_start(s0s, 0, sg0)
                    s_wait(s1d, 1, ss1)
                    unpack(c0 + 5, s1s, s1d)
                    g_start(s1s, 1, sg1)

                @pl.when(g == NIT - 1)
                def _():
                    s_wait(s0d, 0, ss0)
                    s_wait(s1d, 1, ss1)

                return carry

            lax.fori_loop(0, NIT, step, 0)
            plsc.subcore_barrier()
            pltpu.sync_copy(acc.at[pl.ds(r0, RPT)],
                            out_h.at[cid, pl.ds(r0, RPT)])

        return k

    scratch = [
        pltpu.VMEM((NCH, CH), jnp.int32),
        pltpu.VMEM((CH, D), jnp.float32),
        pltpu.SemaphoreType.DMA,
        pltpu.VMEM_SHARED((NPAD, D), jnp.float32),
    ]

    @functools.partial(pl.kernel, out_type=out_type, mesh=mesh,
                       scratch_types=scratch, compiler_params=cparams)
    def k(dst_h, ones_h, z_h, out_h, dst_l, ones_v, sem_s, acc):
        cid = lax.axis_index("c")
        sid = lax.axis_index("s")
        wid = sid * NC + cid
        r0 = sid * RPT
        pltpu.sync_copy(z_h, acc.at[pl.ds(r0, RPT)])
        pltpu.sync_copy(ones_h, ones_v)
        pltpu.sync_copy(dst_h.at[wid], dst_l)
        plsc.subcore_barrier()

        def step(g, carry):
            c0 = g * PD
            sd = [pltpu.async_copy(ones_v, acc.at[dst_l.at[c0 + b]],
                                   sem_s, add=True)
                  for b in range(PD)]
            for b in range(PD):
                sd[b].wait()
            return carry

        lax.fori_loop(0, NCH // PD, step, 0)
        plsc.subcore_barrier()
        pltpu.sync_copy(acc.at[pl.ds(r0, RPT)], out_h.at[cid, pl.ds(r0, RPT)])

    return k


_sc_deg = _sc_prop(8, gather=False)
_sc_prop128 = _sc_prop(D_IN, gather=True, PD=2)
_sc_prop8 = _sc_prop(8, gather=True, PD=4)


def _tc_scale_body(deg_ref, x_ref, dis_ref, xs_ref):
    deg = deg_ref[0][:, :1] + deg_ref[1][:, :1] + 1.0
    dis = lax.rsqrt(deg)
    dis_ref[...] = dis
    xs_ref[...] = x_ref[...] * dis


def _tc_scale(deg2, x):
    return pl.pallas_call(
        _tc_scale_body,
        grid=(GRID,),
        in_specs=[
            pl.BlockSpec((NC, BS, 8), lambda i: (0, i, 0)),
            pl.BlockSpec((BS, D_IN), lambda i: (i, 0)),
        ],
        out_specs=[
            pl.BlockSpec((BS, 1), lambda i: (i, 0)),
            pl.BlockSpec((BS, D_IN), lambda i: (i, 0)),
        ],
        out_shape=[
            jax.ShapeDtypeStruct((NPAD, 1), jnp.float32),
            jax.ShapeDtypeStruct((NPAD, D_IN), jnp.float32),
        ],
    )(deg2, x)


def _tc_mlp_body(y2_ref, xs_ref, dis_ref, w1_ref, b1_ref, w2_ref, wo_ref,
                 qs_ref):
    dis = dis_ref[...]
    y = dis * (y2_ref[0] + y2_ref[1] + xs_ref[...])
    h = jnp.dot(y, w1_ref[...], preferred_element_type=jnp.float32)
    h = jnp.maximum(h + b1_ref[...], 0.0)
    v = jnp.dot(w2_ref[...], wo_ref[...], preferred_element_type=jnp.float32)
    q = jnp.dot(h, v, preferred_element_type=jnp.float32)
    qs_ref[...] = jnp.broadcast_to(dis * q, (BS, 8))


def _tc_mlp(y2, xs, dis, W1, b1, W2, Wo):
    return pl.pallas_call(
        _tc_mlp_body,
        grid=(GRID,),
        in_specs=[
            pl.BlockSpec((NC, BS, D_IN), lambda i: (0, i, 0)),
            pl.BlockSpec((BS, D_IN), lambda i: (i, 0)),
            pl.BlockSpec((BS, 1), lambda i: (i, 0)),
            pl.BlockSpec((D_IN, H), lambda i: (0, 0)),
            pl.BlockSpec((1, H), lambda i: (0, 0)),
            pl.BlockSpec((H, D_EMB), lambda i: (0, 0)),
            pl.BlockSpec((D_EMB, 1), lambda i: (0, 0)),
        ],
        out_specs=pl.BlockSpec((BS, 8), lambda i: (i, 0)),
        out_shape=jax.ShapeDtypeStruct((NPAD, 8), jnp.float32),
    )(y2, xs, dis, W1, b1, W2, Wo)


def _tc_loss_body(p2_ref, qs_ref, dis_ref, lab_ref, mask_ref, b2_ref, wo_ref,
                  bo_ref, out_ref, acc_ref):
    i = pl.program_id(0)
    c = jnp.dot(b2_ref[...], wo_ref[...],
                preferred_element_type=jnp.float32) + bo_ref[...]
    pred = dis_ref[...] * (p2_ref[0][:, :1] + p2_ref[1][:, :1]
                           + qs_ref[:, :1]) + c
    t = lab_ref[...]
    per = jnp.maximum(pred, 0.0) - pred * t + jnp.log1p(jnp.exp(-jnp.abs(pred)))
    m = mask_ref[...]
    bnum = jnp.sum(per * m)
    bden = jnp.sum(m)

    @pl.when(i == 0)
    def _():
        acc_ref[0] = bnum
        acc_ref[1] = bden

    @pl.when(i > 0)
    def _():
        acc_ref[0] = acc_ref[0] + bnum
        acc_ref[1] = acc_ref[1] + bden

    @pl.when(i == pl.num_programs(0) - 1)
    def _():
        out_ref[...] = (acc_ref[0] / acc_ref[1]).reshape(1, 1)


def _tc_loss(p2, qs, dis, lab, maskf, b2, Wo, bo):
    return pl.pallas_call(
        _tc_loss_body,
        grid=(GRID,),
        in_specs=[
            pl.BlockSpec((NC, BS, 8), lambda i: (0, i, 0)),
            pl.BlockSpec((BS, 8), lambda i: (i, 0)),
            pl.BlockSpec((BS, 1), lambda i: (i, 0)),
            pl.BlockSpec((BS, 1), lambda i: (i, 0)),
            pl.BlockSpec((BS, 1), lambda i: (i, 0)),
            pl.BlockSpec((1, D_EMB), lambda i: (0, 0)),
            pl.BlockSpec((D_EMB, 1), lambda i: (0, 0)),
            pl.BlockSpec((1, 1), lambda i: (0, 0)),
        ],
        out_specs=pl.BlockSpec((1, 1), lambda i: (0, 0)),
        out_shape=jax.ShapeDtypeStruct((1, 1), jnp.float32),
        scratch_shapes=[pltpu.SMEM((2,), jnp.float32)],
    )(p2, qs, dis, lab, maskf, b2, Wo, bo)


def kernel(x, edge_index, mask, labels, W1, b1, W2, b2, Wo, bo):
    src = edge_index[0]
    dst = edge_index[1]
    pad = EP - E
    src_p = jnp.concatenate([src, jnp.zeros((pad,), jnp.int32)])
    dst_p = jnp.concatenate([dst, jnp.full((pad,), DUMMY, jnp.int32)])
    pk3 = ((src_p << 16) | dst_p).reshape(NW, NCH, CH)
    dst3 = dst_p.reshape(NW, NCH, CH)
    x_p = jnp.pad(x, ((0, NPAD - N), (0, 0)))
    ones8 = jnp.ones((CH, 8), jnp.float32)
    z8 = jnp.zeros((RPT, 8), jnp.float32)
    z128 = jnp.zeros((RPT, D_IN), jnp.float32)

    deg2 = _sc_deg(dst3, ones8, z8)
    dis, xs = _tc_scale(deg2, x_p)
    y2 = _sc_prop128(pk3, xs, z128)
    qs = _tc_mlp(y2, xs, dis, W1, b1.reshape(1, H), W2, Wo)
    p2 = _sc_prop8(pk3, qs, z8)

    lab_p = jnp.pad(labels, ((0, NPAD - N), (0, 0)))
    mf = jnp.pad(mask.astype(jnp.float32).reshape(N, 1),
                 ((0, NPAD - N), (0, 0)))
    loss = _tc_loss(p2, qs, dis, lab_p, mf, b2.reshape(1, D_EMB), Wo,
                    bo.reshape(1, 1))
    return loss.reshape(())


# speedup vs baseline: 31.1022x; 1.9131x over previous
import functools

import jax
import jax.numpy as jnp
from jax import lax
from jax.experimental import pallas as pl
from jax.experimental.pallas import tpu as pltpu
from jax.experimental.pallas import tpu_sc as plsc

N = 10000
E = 320000
D_IN = 128
H = 256
D_EMB = 128

NPAD = 10240
NC, NS = 2, 16
NW = NC * NS
CH = 128
NCH = 80
EPW = NCH * CH
EP = NW * EPW
DUMMY = 10200
RPT = NPAD // NS
BS = 512
GRID = NPAD // BS


def _sc_prop(D, gather, NPH=1):
    mesh = plsc.VectorSubcoreMesh(core_axis_name="c", subcore_axis_name="s")
    out_type = jax.ShapeDtypeStruct((NPH, NC, NPAD, D), jnp.float32)
    cparams = pltpu.CompilerParams(use_tc_tiling_on_sc=False)

    if gather:
        scratch = [
            pltpu.VMEM((NCH, CH), jnp.int32),
            [pltpu.VMEM((CH,), jnp.int32) for _ in range(4)],
            pltpu.VMEM((2, CH, D), jnp.float32),
            [pltpu.SemaphoreType.DMA for _ in range(4)],
            pltpu.VMEM_SHARED((NPAD, D), jnp.float32),
            pltpu.VMEM_SHARED((NPAD, D), jnp.float32),
        ]
        NIT = NCH // 4

        @functools.partial(pl.kernel, out_type=out_type, mesh=mesh,
                           scratch_types=scratch, compiler_params=cparams)
        def k(pk_h, tab_h, z_h, out_h, pk_l, slots, rows, sems, tabs, acc):
            cid = lax.axis_index("c")
            sid = lax.axis_index("s")
            wid = sid * NC + cid
            r0 = sid * RPT
            s0s, s0d, s1s, s1d = slots
            sg0, sg1, ss0, ss1 = sems
            pltpu.sync_copy(pk_h.at[wid], pk_l)

            def unpack(c, ssrc, sdst):
                for j in range(CH // 16):
                    v = pk_l[c, pl.ds(j * 16, 16)]
                    ssrc[pl.ds(j * 16, 16)] = lax.shift_right_logical(v, 16)
                    sdst[pl.ds(j * 16, 16)] = lax.bitwise_and(v, 0xFFFF)

            def g_start(ssrc, b, sem):
                pltpu.async_copy(tabs.at[ssrc], rows.at[b], sem)

            def g_wait(ssrc, b, sem):
                pltpu.make_async_copy(tabs.at[ssrc], rows.at[b], sem).wait()

            def s_start(sdst, b, sem):
                pltpu.async_copy(rows.at[b], acc.at[sdst], sem, add=True)

            def s_wait(sdst, b, sem):
                pltpu.make_async_copy(rows.at[b], acc.at[sdst], sem).wait()

            for p in range(NPH):
                pltpu.sync_copy(z_h, acc.at[pl.ds(r0, RPT)])
                pltpu.sync_copy(tab_h.at[p, pl.ds(r0, RPT)],
                                tabs.at[pl.ds(r0, RPT)])
                plsc.subcore_barrier()

                unpack(0, s0s, s0d)
                unpack(1, s1s, s1d)
                g_start(s0s, 0, sg0)
                g_start(s1s, 1, sg1)

                def step(g, carry):
                    c0 = g * 4
                    g_wait(s0s, 0, sg0)
                    s_start(s0d, 0, ss0)
                    g_wait(s1s, 1, sg1)
                    s_start(s1d, 1, ss1)
                    s_wait(s0d, 0, ss0)
                    unpack(c0 + 2, s0s, s0d)
                    g_start(s0s, 0, sg0)
                    s_wait(s1d, 1, ss1)
                    unpack(c0 + 3, s1s, s1d)
                    g_start(s1s, 1, sg1)
                    g_wait(s0s, 0, sg0)
                    s_start(s0d, 0, ss0)
                    g_wait(s1s, 1, sg1)
                    s_start(s1d, 1, ss1)

                    @pl.when(g < NIT - 1)
                    def _():
                        s_wait(s0d, 0, ss0)
                        unpack(c0 + 4, s0s, s0d)
                        g_start(s0s, 0, sg0)
                        s_wait(s1d, 1, ss1)
                        unpack(c0 + 5, s1s, s1d)
                        g_start(s1s, 1, sg1)

                    @pl.when(g == NIT - 1)
                    def _():
                        s_wait(s0d, 0, ss0)
                        s_wait(s1d, 1, ss1)

                    return carry

                lax.fori_loop(0, NIT, step, 0)
                plsc.subcore_barrier()
                pltpu.sync_copy(acc.at[pl.ds(r0, RPT)],
                                out_h.at[p, cid, pl.ds(r0, RPT)])

        return k

    PDG = 8
    scratch = [
        pltpu.VMEM((NCH, CH), jnp.int32),
        pltpu.VMEM((CH, D), jnp.float32),
        pltpu.SemaphoreType.DMA,
        pltpu.VMEM_SHARED((NPAD, D), jnp.float32),
    ]

    @functools.partial(pl.kernel, out_type=out_type, mesh=mesh,
                       scratch_types=scratch, compiler_params=cparams)
    def k(dst_h, ones_h, z_h, out_h, dst_l, ones_v, sem_s, acc):
        cid = lax.axis_index("c")
        sid = lax.axis_index("s")
        wid = sid * NC + cid
        r0 = sid * RPT
        pltpu.sync_copy(z_h, acc.at[pl.ds(r0, RPT)])
        pltpu.sync_copy(ones_h, ones_v)
        pltpu.sync_copy(dst_h.at[wid], dst_l)
        plsc.subcore_barrier()

        def step(g, carry):
            c0 = g * PDG
            sd = [pltpu.async_copy(ones_v, acc.at[dst_l.at[c0 + b]],
                                   sem_s, add=True)
                  for b in range(PDG)]
            for b in range(PDG):
                sd[b].wait()
            return carry

        lax.fori_loop(0, NCH // PDG, step, 0)
        plsc.subcore_barrier()
        pltpu.sync_copy(acc.at[pl.ds(r0, RPT)],
                        out_h.at[0, cid, pl.ds(r0, RPT)])

    return k


_sc_deg = _sc_prop(8, gather=False)
_sc_prop64 = _sc_prop(64, gather=True, NPH=2)
_sc_prop8 = _sc_prop(8, gather=True, NPH=1)


def _tc_scale_body(deg_ref, x_ref, dis_ref, xs_ref):
    deg = deg_ref[0, 0][:, :1] + deg_ref[0, 1][:, :1] + 1.0
    dis = lax.rsqrt(deg)
    dis_ref[...] = dis
    xs = x_ref[...] * dis
    xs_ref[0] = xs[:, :64]
    xs_ref[1] = xs[:, 64:]


def _tc_scale(deg2, x):
    return pl.pallas_call(
        _tc_scale_body,
        grid=(GRID,),
        in_specs=[
            pl.BlockSpec((1, NC, BS, 8), lambda i: (0, 0, i, 0)),
            pl.BlockSpec((BS, D_IN), lambda i: (i, 0)),
        ],
        out_specs=[
            pl.BlockSpec((BS, 1), lambda i: (i, 0)),
            pl.BlockSpec((2, BS, 64), lambda i: (0, i, 0)),
        ],
        out_shape=[
            jax.ShapeDtypeStruct((NPAD, 1), jnp.float32),
            jax.ShapeDtypeStruct((2, NPAD, 64), jnp.float32),
        ],
    )(deg2, x)


def _tc_mlp_body(y2_ref, xs_ref, dis_ref, w1_ref, b1_ref, w2_ref, wo_ref,
                 qs_ref):
    dis = dis_ref[...]
    ya = y2_ref[0, 0] + y2_ref[0, 1] + xs_ref[0]
    yb = y2_ref[1, 0] + y2_ref[1, 1] + xs_ref[1]
    y = dis * jnp.concatenate([ya, yb], axis=1)
    h = jnp.dot(y, w1_ref[...], preferred_element_type=jnp.float32)
    h = jnp.maximum(h + b1_ref[...], 0.0)
    v = jnp.dot(w2_ref[...], wo_ref[...], preferred_element_type=jnp.float32)
    q = jnp.dot(h, v, preferred_element_type=jnp.float32)
    qs_ref[...] = jnp.broadcast_to(dis * q, (BS, 8))


def _tc_mlp(y2, xs, dis, W1, b1, W2, Wo):
    return pl.pallas_call(
        _tc_mlp_body,
        grid=(GRID,),
        in_specs=[
            pl.BlockSpec((2, NC, BS, 64), lambda i: (0, 0, i, 0)),
            pl.BlockSpec((2, BS, 64), lambda i: (0, i, 0)),
            pl.BlockSpec((BS, 1), lambda i: (i, 0)),
            pl.BlockSpec((D_IN, H), lambda i: (0, 0)),
            pl.BlockSpec((1, H), lambda i: (0, 0)),
            pl.BlockSpec((H, D_EMB), lambda i: (0, 0)),
            pl.BlockSpec((D_EMB, 1), lambda i: (0, 0)),
        ],
        out_specs=pl.BlockSpec((BS, 8), lambda i: (i, 0)),
        out_shape=jax.ShapeDtypeStruct((NPAD, 8), jnp.float32),
    )(y2, xs, dis, W1, b1, W2, Wo)


def _tc_loss_body(p2_ref, qs_ref, dis_ref, lab_ref, mask_ref, b2_ref, wo_ref,
                  bo_ref, out_ref, acc_ref):
    i = pl.program_id(0)
    c = jnp.dot(b2_ref[...], wo_ref[...],
                preferred_element_type=jnp.float32) + bo_ref[...]
    pred = dis_ref[...] * (p2_ref[0, 0][:, :1] + p2_ref[0, 1][:, :1]
                           + qs_ref[:, :1]) + c
    t = lab_ref[...]
    per = jnp.maximum(pred, 0.0) - pred * t + jnp.log1p(jnp.exp(-jnp.abs(pred)))
    m = mask_ref[...]
    bnum = jnp.sum(per * m)
    bden = jnp.sum(m)

    @pl.when(i == 0)
    def _():
        acc_ref[0] = bnum
        acc_ref[1] = bden

    @pl.when(i > 0)
    def _():
        acc_ref[0] = acc_ref[0] + bnum
        acc_ref[1] = acc_ref[1] + bden

    @pl.when(i == pl.num_programs(0) - 1)
    def _():
        out_ref[...] = (acc_ref[0] / acc_ref[1]).reshape(1, 1)


def _tc_loss(p2, qs, dis, lab, maskf, b2, Wo, bo):
    return pl.pallas_call(
        _tc_loss_body,
        grid=(GRID,),
        in_specs=[
            pl.BlockSpec((1, NC, BS, 8), lambda i: (0, 0, i, 0)),
            pl.BlockSpec((BS, 8), lambda i: (i, 0)),
            pl.BlockSpec((BS, 1), lambda i: (i, 0)),
            pl.BlockSpec((BS, 1), lambda i: (i, 0)),
            pl.BlockSpec((BS, 1), lambda i: (i, 0)),
            pl.BlockSpec((1, D_EMB), lambda i: (0, 0)),
            pl.BlockSpec((D_EMB, 1), lambda i: (0, 0)),
            pl.BlockSpec((1, 1), lambda i: (0, 0)),
        ],
        out_specs=pl.BlockSpec((1, 1), lambda i: (0, 0)),
        out_shape=jax.ShapeDtypeStruct((1, 1), jnp.float32),
        scratch_shapes=[pltpu.SMEM((2,), jnp.float32)],
    )(p2, qs, dis, lab, maskf, b2, Wo, bo)


def kernel(x, edge_index, mask, labels, W1, b1, W2, b2, Wo, bo):
    src = edge_index[0]
    dst = edge_index[1]
    pad = EP - E
    src_p = jnp.concatenate([src, jnp.zeros((pad,), jnp.int32)])
    dst_p = jnp.concatenate([dst, jnp.full((pad,), DUMMY, jnp.int32)])
    pk3 = ((src_p << 16) | dst_p).reshape(NW, NCH, CH)
    dst3 = dst_p.reshape(NW, NCH, CH)
    x_p = jnp.pad(x, ((0, NPAD - N), (0, 0)))
    ones8 = jnp.ones((CH, 8), jnp.float32)
    z8 = jnp.zeros((RPT, 8), jnp.float32)
    z64 = jnp.zeros((RPT, 64), jnp.float32)

    deg2 = _sc_deg(dst3, ones8, z8)
    dis, xs2 = _tc_scale(deg2, x_p)
    y2 = _sc_prop64(pk3, xs2, z64)
    qs = _tc_mlp(y2, xs2, dis, W1, b1.reshape(1, H), W2, Wo)
    p2 = _sc_prop8(pk3, qs.reshape(1, NPAD, 8), z8)

    lab_p = jnp.pad(labels, ((0, NPAD - N), (0, 0)))
    mf = jnp.pad(mask.astype(jnp.float32).reshape(N, 1),
                 ((0, NPAD - N), (0, 0)))
    loss = _tc_loss(p2, qs, dis, lab_p, mf, b2.reshape(1, D_EMB), Wo,
                    bo.reshape(1, 1))
    return loss.reshape(())


# speedup vs baseline: 33.0951x; 1.0641x over previous
import functools

import jax
import jax.numpy as jnp
from jax import lax
from jax.experimental import pallas as pl
from jax.experimental.pallas import tpu as pltpu
from jax.experimental.pallas import tpu_sc as plsc

N = 10000
E = 320000
D_IN = 128
H = 256
D_EMB = 128

NPAD = 10240
NC, NS = 2, 16
NW = NC * NS
CH = 128
NCH = 80
EPW = NCH * CH
EP = NW * EPW
DUMMY = 10200
RPT = NPAD // NS
BS = 1024
GRID = NPAD // BS
BSN = 1000
GRIDN = N // BSN


def _sc_prop(D, gather, NPH=1):
    mesh = plsc.VectorSubcoreMesh(core_axis_name="c", subcore_axis_name="s")
    out_type = jax.ShapeDtypeStruct((NPH, NC, NPAD, D), jnp.float32)
    cparams = pltpu.CompilerParams(use_tc_tiling_on_sc=False)

    if gather:
        scratch = [
            pltpu.VMEM((NCH, CH), jnp.int32),
            [pltpu.VMEM((CH,), jnp.int32) for _ in range(4)],
            pltpu.VMEM((2, CH, D), jnp.float32),
            [pltpu.SemaphoreType.DMA for _ in range(4)],
            pltpu.VMEM_SHARED((NPAD, D), jnp.float32),
            pltpu.VMEM_SHARED((NPAD, D), jnp.float32),
        ]
        NIT = NCH // 4

        @functools.partial(pl.kernel, out_type=out_type, mesh=mesh,
                           scratch_types=scratch, compiler_params=cparams)
        def k(pk_h, tab_h, z_h, out_h, pk_l, slots, rows, sems, tabs, acc):
            cid = lax.axis_index("c")
            sid = lax.axis_index("s")
            wid = sid * NC + cid
            r0 = sid * RPT
            s0s, s0d, s1s, s1d = slots
            sg0, sg1, ss0, ss1 = sems
            pltpu.sync_copy(pk_h.at[wid], pk_l)

            def unpack(c, ssrc, sdst):
                for j in range(CH // 16):
                    v = pk_l[c, pl.ds(j * 16, 16)]
                    ssrc[pl.ds(j * 16, 16)] = lax.shift_right_logical(v, 16)
                    sdst[pl.ds(j * 16, 16)] = lax.bitwise_and(v, 0xFFFF)

            def g_start(ssrc, b, sem):
                pltpu.async_copy(tabs.at[ssrc], rows.at[b], sem)

            def g_wait(ssrc, b, sem):
                pltpu.make_async_copy(tabs.at[ssrc], rows.at[b], sem).wait()

            def s_start(sdst, b, sem):
                pltpu.async_copy(rows.at[b], acc.at[sdst], sem, add=True)

            def s_wait(sdst, b, sem):
                pltpu.make_async_copy(rows.at[b], acc.at[sdst], sem).wait()

            for p in range(NPH):
                pltpu.sync_copy(z_h, acc.at[pl.ds(r0, RPT)])
                pltpu.sync_copy(tab_h.at[p, pl.ds(r0, RPT)],
                                tabs.at[pl.ds(r0, RPT)])
                plsc.subcore_barrier()

                unpack(0, s0s, s0d)
                unpack(1, s1s, s1d)
                g_start(s0s, 0, sg0)
                g_start(s1s, 1, sg1)

                def step(g, carry):
                    c0 = g * 4
                    g_wait(s0s, 0, sg0)
                    s_start(s0d, 0, ss0)
                    g_wait(s1s, 1, sg1)
                    s_start(s1d, 1, ss1)
                    s_wait(s0d, 0, ss0)
                    unpack(c0 + 2, s0s, s0d)
                    g_start(s0s, 0, sg0)
                    s_wait(s1d, 1, ss1)
                    unpack(c0 + 3, s1s, s1d)
                    g_start(s1s, 1, sg1)
                    g_wait(s0s, 0, sg0)
                    s_start(s0d, 0, ss0)
                    g_wait(s1s, 1, sg1)
                    s_start(s1d, 1, ss1)

                    @pl.when(g < NIT - 1)
                    def _():
                        s_wait(s0d, 0, ss0)
                        unpack(c0 + 4, s0s, s0d)
                        g_start(s0s, 0, sg0)
                        s_wait(s1d, 1, ss1)
                        unpack(c0 + 5, s1s, s1d)
                        g_start(s1s, 1, sg1)

                    @pl.when(g == NIT - 1)
                    def _():
                        s_wait(s0d, 0, ss0)
                        s_wait(s1d, 1, ss1)

                    return carry

                lax.fori_loop(0, NIT, step, 0)
                plsc.subcore_barrier()
                pltpu.sync_copy(acc.at[pl.ds(r0, RPT)],
                                out_h.at[p, cid, pl.ds(r0, RPT)])

        return k

    PDG = 8
    scratch = [
        pltpu.VMEM((NCH, CH), jnp.int32),
        pltpu.VMEM((CH, D), jnp.float32),
        pltpu.SemaphoreType.DMA,
        pltpu.VMEM_SHARED((NPAD, D), jnp.float32),
    ]

    @functools.partial(pl.kernel, out_type=out_type, mesh=mesh,
                       scratch_types=scratch, compiler_params=cparams)
    def k(dst_h, ones_h, z_h, out_h, dst_l, ones_v, sem_s, acc):
        cid = lax.axis_index("c")
        sid = lax.axis_index("s")
        wid = sid * NC + cid
        r0 = sid * RPT
        pltpu.sync_copy(z_h, acc.at[pl.ds(r0, RPT)])
        pltpu.sync_copy(ones_h, ones_v)
        pltpu.sync_copy(dst_h.at[wid], dst_l)
        plsc.subcore_barrier()

        def step(g, carry):
            c0 = g * PDG
            sd = [pltpu.async_copy(ones_v, acc.at[dst_l.at[c0 + b]],
                                   sem_s, add=True)
                  for b in range(PDG)]
            for b in range(PDG):
                sd[b].wait()
            return carry

        lax.fori_loop(0, NCH // PDG, step, 0)
        plsc.subcore_barrier()
        pltpu.sync_copy(acc.at[pl.ds(r0, RPT)],
                        out_h.at[0, cid, pl.ds(r0, RPT)])

    return k


_sc_deg = _sc_prop(8, gather=False)
_sc_prop64 = _sc_prop(64, gather=True, NPH=2)
_sc_prop8 = _sc_prop(8, gather=True, NPH=1)


def _tc_scale_body(deg_ref, x_ref, dis_ref, xs_ref):
    deg = deg_ref[0, 0][:, :1] + deg_ref[0, 1][:, :1] + 1.0
    dis = lax.rsqrt(deg)
    dis_ref[...] = dis
    xs = x_ref[...] * dis
    xs_ref[0] = xs[:, :64]
    xs_ref[1] = xs[:, 64:]


def _tc_scale(deg2, x):
    return pl.pallas_call(
        _tc_scale_body,
        grid=(GRIDN,),
        in_specs=[
            pl.BlockSpec((1, NC, BSN, 8), lambda i: (0, 0, i, 0)),
            pl.BlockSpec((BSN, D_IN), lambda i: (i, 0)),
        ],
        out_specs=[
            pl.BlockSpec((BSN, 1), lambda i: (i, 0)),
            pl.BlockSpec((2, BSN, 64), lambda i: (0, i, 0)),
        ],
        out_shape=[
            jax.ShapeDtypeStruct((NPAD, 1), jnp.float32),
            jax.ShapeDtypeStruct((2, NPAD, 64), jnp.float32),
        ],
    )(deg2, x)


def _tc_mlp_body(y2_ref, xs_ref, dis_ref, w1_ref, b1_ref, w2_ref, wo_ref,
                 qs_ref):
    dis = dis_ref[...]
    ya = y2_ref[0, 0] + y2_ref[0, 1] + xs_ref[0]
    yb = y2_ref[1, 0] + y2_ref[1, 1] + xs_ref[1]
    y = dis * jnp.concatenate([ya, yb], axis=1)
    h = jnp.dot(y, w1_ref[...], preferred_element_type=jnp.float32)
    h = jnp.maximum(h + b1_ref[...], 0.0)
    v = jnp.dot(w2_ref[...], wo_ref[...], preferred_element_type=jnp.float32)
    q = jnp.dot(h, v, preferred_element_type=jnp.float32)
    qs_ref[0] = jnp.broadcast_to(dis * q, (BS, 8))


def _tc_mlp(y2, xs, dis, W1, b1, W2, Wo):
    return pl.pallas_call(
        _tc_mlp_body,
        grid=(GRID,),
        in_specs=[
            pl.BlockSpec((2, NC, BS, 64), lambda i: (0, 0, i, 0)),
            pl.BlockSpec((2, BS, 64), lambda i: (0, i, 0)),
            pl.BlockSpec((BS, 1), lambda i: (i, 0)),
            pl.BlockSpec((D_IN, H), lambda i: (0, 0)),
            pl.BlockSpec((1, H), lambda i: (0, 0)),
            pl.BlockSpec((H, D_EMB), lambda i: (0, 0)),
            pl.BlockSpec((D_EMB, 1), lambda i: (0, 0)),
        ],
        out_specs=pl.BlockSpec((1, BS, 8), lambda i: (0, i, 0)),
        out_shape=jax.ShapeDtypeStruct((1, NPAD, 8), jnp.float32),
    )(y2, xs, dis, W1, b1, W2, Wo)


def _tc_loss_body(p2_ref, qs_ref, dis_ref, lab_ref, mask_ref, b2_ref, wo_ref,
                  bo_ref, out_ref, acc_ref):
    i = pl.program_id(0)
    c = jnp.dot(b2_ref[...], wo_ref[...],
                preferred_element_type=jnp.float32) + bo_ref[...]
    pred = dis_ref[...] * (p2_ref[0, 0][:, :1] + p2_ref[0, 1][:, :1]
                           + qs_ref[0][:, :1]) + c
    t = lab_ref[...]
    per = jnp.maximum(pred, 0.0) - pred * t + jnp.log1p(jnp.exp(-jnp.abs(pred)))
    m = mask_ref[...]
    bnum = jnp.sum(jnp.where(m > 0, per, 0.0))
    bden = jnp.sum(m)

    @pl.when(i == 0)
    def _():
        acc_ref[0] = bnum
        acc_ref[1] = bden

    @pl.when(i > 0)
    def _():
        acc_ref[0] = acc_ref[0] + bnum
        acc_ref[1] = acc_ref[1] + bden

    @pl.when(i == pl.num_programs(0) - 1)
    def _():
        out_ref[...] = (acc_ref[0] / acc_ref[1]).reshape(1, 1)


def _tc_loss(p2, qs, dis, lab, maskf, b2, Wo, bo):
    return pl.pallas_call(
        _tc_loss_body,
        grid=(GRIDN,),
        in_specs=[
            pl.BlockSpec((1, NC, BSN, 8), lambda i: (0, 0, i, 0)),
            pl.BlockSpec((1, BSN, 8), lambda i: (0, i, 0)),
            pl.BlockSpec((BSN, 1), lambda i: (i, 0)),
            pl.BlockSpec((BSN, 1), lambda i: (i, 0)),
            pl.BlockSpec((BSN, 1), lambda i: (i, 0)),
            pl.BlockSpec((1, D_EMB), lambda i: (0, 0)),
            pl.BlockSpec((D_EMB, 1), lambda i: (0, 0)),
            pl.BlockSpec((1, 1), lambda i: (0, 0)),
        ],
        out_specs=pl.BlockSpec((1, 1), lambda i: (0, 0)),
        out_shape=jax.ShapeDtypeStruct((1, 1), jnp.float32),
        scratch_shapes=[pltpu.SMEM((2,), jnp.float32)],
    )(p2, qs, dis, lab, maskf, b2, Wo, bo)


def kernel(x, edge_index, mask, labels, W1, b1, W2, b2, Wo, bo):
    src = edge_index[0]
    dst = edge_index[1]
    pad = EP - E
    src_p = jnp.concatenate([src, jnp.zeros((pad,), jnp.int32)])
    dst_p = jnp.concatenate([dst, jnp.full((pad,), DUMMY, jnp.int32)])
    pk3 = ((src_p << 16) | dst_p).reshape(NW, NCH, CH)
    dst3 = dst_p.reshape(NW, NCH, CH)
    ones8 = jnp.ones((CH, 8), jnp.float32)
    z8 = jnp.zeros((RPT, 8), jnp.float32)
    z64 = jnp.zeros((RPT, 64), jnp.float32)

    deg2 = _sc_deg(dst3, ones8, z8)
    dis, xs2 = _tc_scale(deg2, x)
    y2 = _sc_prop64(pk3, xs2, z64)
    qs = _tc_mlp(y2, xs2, dis, W1, b1.reshape(1, H), W2, Wo)
    p2 = _sc_prop8(pk3, qs, z8)

    mf = mask.astype(jnp.float32).reshape(N, 1)
    loss = _tc_loss(p2, qs, dis, labels, mf, b2.reshape(1, D_EMB), Wo,
                    bo.reshape(1, 1))
    return loss.reshape(())
